# baseline (speedup 1.0000x reference)
import jax
import jax.numpy as jnp
from jax import lax
from jax.experimental import pallas as pl
from jax.experimental.pallas import tpu as pltpu
from jax.experimental.pallas import tpu_sc as plsc

_BATCH = 4
_NUM_AGENTS = 4
_D0 = 128
_D1 = 64
_DOUT = _D0 + _D1
_ROWS = _BATCH * _NUM_AGENTS
_LANES = 16


def _root_gather_kernel(qs0_hbm, qs1_hbm, out_hbm):
    r = lax.axis_index("s")

    node = lax.iota(jnp.int32, _LANES)
    is_init = node == 0
    used = is_init
    agent_idx = jnp.where(is_init, _NUM_AGENTS - 1, 0)
    horizon = jnp.zeros((_LANES,), jnp.int32)
    obs0 = jnp.where(is_init, 0, -1)
    obs1 = jnp.where(is_init, 0, -1)

    mask = (
        used
        & (agent_idx == _NUM_AGENTS - 1)
        & (horizon == 0)
        & jnp.logical_not((obs0 == -1) & (obs1 == -1))
    )
    ridx = jnp.min(jnp.where(mask, node, jnp.int32(2**30)))

    src = ridx * _ROWS + r
    pltpu.sync_copy(qs0_hbm.at[src], out_hbm.at[r, pl.ds(0, _D0)])
    pltpu.sync_copy(qs1_hbm.at[src], out_hbm.at[r, pl.ds(_D0, _D1)])


@jax.jit
def kernel(qs_0, qs_1):
    qs0_rows = qs_0.reshape(_ROWS, _D0)
    qs1_rows = qs_1.reshape(_ROWS, _D1)
    mesh = plsc.VectorSubcoreMesh(
        core_axis_name="c", subcore_axis_name="s", num_cores=1
    )
    out = pl.kernel(
        _root_gather_kernel,
        mesh=mesh,
        out_type=jax.ShapeDtypeStruct((_ROWS, _DOUT), jnp.float32),
        compiler_params=pltpu.CompilerParams(
            needs_layout_passes=False,
            use_tc_tiling_on_sc=False,
            skip_device_barrier=True,
        ),
    )(qs0_rows, qs1_rows)
    return out.reshape(_BATCH, _NUM_AGENTS, 1, _DOUT)


# speedup vs baseline: 1.0664x; 1.0664x over previous
import jax
import jax.numpy as jnp
from jax import lax
from jax.experimental import pallas as pl
from jax.experimental.pallas import tpu as pltpu
from jax.experimental.pallas import tpu_sc as plsc

_BATCH = 4
_NUM_AGENTS = 4
_D0 = 128
_D1 = 64
_DOUT = _D0 + _D1
_ROWS = _BATCH * _NUM_AGENTS
_LANES = 16


def _root_gather_kernel(qs0_hbm, qs1_hbm, out_hbm, row_v, sem0, sem1):
    r = lax.axis_index("s")

    node = lax.iota(jnp.int32, _LANES)
    is_init = node == 0
    used = is_init
    agent_idx = jnp.where(is_init, _NUM_AGENTS - 1, 0)
    horizon = jnp.zeros((_LANES,), jnp.int32)
    obs0 = jnp.where(is_init, 0, -1)
    obs1 = jnp.where(is_init, 0, -1)

    mask = (
        used
        & (agent_idx == _NUM_AGENTS - 1)
        & (horizon == 0)
        & jnp.logical_not((obs0 == -1) & (obs1 == -1))
    )
    ridx = jnp.min(jnp.where(mask, node, jnp.int32(2**30)))

    src = ridx * _ROWS + r
    cp0 = pltpu.async_copy(qs0_hbm.at[src], row_v.at[pl.ds(0, _D0)], sem0)
    cp1 = pltpu.async_copy(qs1_hbm.at[src], row_v.at[pl.ds(_D0, _D1)], sem1)
    cp0.wait()
    cp1.wait()
    pltpu.sync_copy(row_v, out_hbm.at[r])


@jax.jit
def kernel(qs_0, qs_1):
    qs0_rows = qs_0.reshape(_ROWS, _D0)
    qs1_rows = qs_1.reshape(_ROWS, _D1)
    mesh = plsc.VectorSubcoreMesh(
        core_axis_name="c", subcore_axis_name="s", num_cores=1
    )
    out = pl.kernel(
        _root_gather_kernel,
        mesh=mesh,
        out_type=jax.ShapeDtypeStruct((_ROWS, _DOUT), jnp.float32),
        scratch_types=[
            pltpu.VMEM((_DOUT,), jnp.float32),
            pltpu.SemaphoreType.DMA,
            pltpu.SemaphoreType.DMA,
        ],
        compiler_params=pltpu.CompilerParams(
            needs_layout_passes=False,
            use_tc_tiling_on_sc=False,
        ),
    )(qs0_rows, qs1_rows)
    return out.reshape(_BATCH, _NUM_AGENTS, 1, _DOUT)


# speedup vs baseline: 1.1350x; 1.0644x over previous
import jax
import jax.numpy as jnp
from jax import lax
from jax.experimental import pallas as pl
from jax.experimental.pallas import tpu as pltpu
from jax.experimental.pallas import tpu_sc as plsc

_BATCH = 4
_NUM_AGENTS = 4
_D0 = 128
_D1 = 64
_DOUT = _D0 + _D1
_ROWS = _BATCH * _NUM_AGENTS
_LANES = 16


def _root_gather_scs(qs0_hbm, qs1_hbm, out_hbm, sem):
    def body(n, best):
        used = n == 0
        agent_idx = jnp.where(n == 0, _NUM_AGENTS - 1, 0)
        horizon = jnp.int32(0)
        obs0 = jnp.where(n == 0, 0, -1)
        obs1 = jnp.where(n == 0, 0, -1)
        m = (
            used
            & (agent_idx == _NUM_AGENTS - 1)
            & (horizon == 0)
            & jnp.logical_not((obs0 == -1) & (obs1 == -1))
        )
        return jnp.where(m & (n < best), n, best)

    ridx = lax.fori_loop(0, _LANES, body, jnp.int32(2**30))

    copies = []
    for r in range(_ROWS):
        src = ridx * _ROWS + r
        copies.append(
            pltpu.async_copy(qs0_hbm.at[src], out_hbm.at[r, pl.ds(0, _D0)], sem)
        )
        copies.append(
            pltpu.async_copy(qs1_hbm.at[src], out_hbm.at[r, pl.ds(_D0, _D1)], sem)
        )
    for c in copies:
        c.wait()


@jax.jit
def kernel(qs_0, qs_1):
    qs0_rows = qs_0.reshape(_ROWS, _D0)
    qs1_rows = qs_1.reshape(_ROWS, _D1)
    mesh = plsc.ScalarSubcoreMesh(axis_name="c", num_cores=1)
    out = pl.kernel(
        _root_gather_scs,
        mesh=mesh,
        out_type=jax.ShapeDtypeStruct((_ROWS, _DOUT), jnp.float32),
        scratch_types=[pltpu.SemaphoreType.DMA],
        compiler_params=pltpu.CompilerParams(
            needs_layout_passes=False,
            use_tc_tiling_on_sc=False,
        ),
    )(qs0_rows, qs1_rows)
    return out.reshape(_BATCH, _NUM_AGENTS, 1, _DOUT)
